# baseline (speedup 1.0000x reference)
import math

import jax
import jax.numpy as jnp
from jax.experimental import pallas as pl
from jax.experimental.pallas import tpu as pltpu

FEATS = 64
NPF = FEATS // 2
H = 64
W = 64
B = 4
NUM_OFFSETS = 7
_TEMPERATURE = 10000.0
_SCALE = 2.0 * math.pi
_EPS = 1e-6


def _body(pos_ref, offt_ref, out_ref, sel_ref):
    i = pl.program_id(0)

    @pl.when(i == 0)
    def _compute():
        hw = H * W
        c = jax.lax.broadcasted_iota(jnp.int32, (FEATS, hw), 0)
        col = jax.lax.broadcasted_iota(jnp.int32, (FEATS, hw), 1)
        y = col // W
        x = col - y * W
        is_y = c < NPF
        cm = jnp.where(is_y, c, c - NPF)
        k = cm // 2
        e = (jnp.where(is_y, y, x).astype(jnp.float32) + 1.0) * (
            _SCALE / (float(H) + _EPS)
        )
        inv_d = jnp.exp(k.astype(jnp.float32) * (-math.log(_TEMPERATURE) * 2.0 / NPF))
        arg = e * inv_d
        pe = jnp.where(cm % 2 == 0, jnp.sin(arg), jnp.cos(arg))
        pos = pos_ref[0, 0]
        lane = jax.lax.broadcasted_iota(jnp.int32, (FEATS, NUM_OFFSETS), 1)
        off = jnp.sum(
            jnp.where(lane == pos, offt_ref[...], 0.0), axis=1, keepdims=True
        )
        sel_ref[...] = pe + off

    out_ref[0] = sel_ref[...]


def kernel(base_memposenc_offsets, imagelike_shape_bchw, position_offset):
    del imagelike_shape_bchw
    offt = base_memposenc_offsets.reshape(NUM_OFFSETS, FEATS).T
    pos = jnp.asarray(position_offset, jnp.int32).reshape(1, 1)
    out = pl.pallas_call(
        _body,
        grid=(B,),
        in_specs=[
            pl.BlockSpec(memory_space=pltpu.SMEM),
            pl.BlockSpec(memory_space=pltpu.VMEM),
        ],
        out_specs=pl.BlockSpec((1, FEATS, H * W), lambda i: (i, 0, 0)),
        out_shape=jax.ShapeDtypeStruct((B, FEATS, H * W), jnp.float32),
        scratch_shapes=[pltpu.VMEM((FEATS, H * W), jnp.float32)],
    )(pos, offt)
    return out.reshape(B, FEATS, H, W)
